# Initial kernel scaffold; baseline (speedup 1.0000x reference)
import functools

import jax
import jax.numpy as jnp
from jax import lax
from jax.experimental import pallas as pl
from jax.experimental.pallas import tpu as pltpu
from jax.experimental.pallas import tpu_sc as plsc

NPAD = 10240
GSEG = 128
BLK = 512


def _sc_count(dstR, nc, ns, rpw):
    mesh = plsc.VectorSubcoreMesh(core_axis_name="c", subcore_axis_name="s")
    rows_per_tile = NPAD // ns

    @functools.partial(
        pl.kernel,
        out_type=jax.ShapeDtypeStruct((nc, NPAD), jnp.float32),
        mesh=mesh,
        scratch_types=[
            pltpu.VMEM((rpw, 128), jnp.int32),
            pltpu.VMEM((128,), jnp.float32),
            pltpu.VMEM_SHARED((NPAD,), jnp.float32),
        ],
    )
    def k(dst_hbm, out_hbm, dst_v, val_v, acc_sh):
        c = lax.axis_index("c")
        s = lax.axis_index("s")
        wid = c * ns + s
        zero16 = jnp.zeros((16,), jnp.float32)
        for j in range(128 // 16):
            val_v[pl.ds(j * 16, 16)] = zero16
        for j in range(rows_per_tile // 128):
            pltpu.sync_copy(val_v, acc_sh.at[pl.ds(s * rows_per_tile + j * 128, 128)])
        plsc.subcore_barrier()
        one16 = jnp.ones((16,), jnp.float32)
        for j in range(128 // 16):
            val_v[pl.ds(j * 16, 16)] = one16
        pltpu.sync_copy(dst_hbm.at[pl.ds(wid * rpw, rpw)], dst_v)
        for j in range(rpw):
            pltpu.sync_copy(val_v, acc_sh.at[dst_v.at[j]], add=True)
        plsc.subcore_barrier()
        pltpu.sync_copy(acc_sh.at[pl.ds(s * rows_per_tile, rows_per_tile)],
                        out_hbm.at[c, pl.ds(s * rows_per_tile, rows_per_tile)])

    return k(dstR)


def _sc_scatter(z, srcR, dstR, nc, ns, rpw, feat):
    mesh = plsc.VectorSubcoreMesh(core_axis_name="c", subcore_axis_name="s")
    rows_per_tile = NPAD // ns

    @functools.partial(
        pl.kernel,
        out_type=jax.ShapeDtypeStruct((nc, NPAD, feat), jnp.float32),
        mesh=mesh,
        scratch_types=[
            pltpu.VMEM((rpw, 128), jnp.int32),
            pltpu.VMEM((rpw, 128), jnp.int32),
            pltpu.VMEM((128, feat), jnp.float32),
            pltpu.VMEM_SHARED((NPAD, feat), jnp.float32),
        ],
    )
    def k(z_hbm, src_hbm, dst_hbm, out_hbm, src_v, dst_v, buf, acc_sh):
        c = lax.axis_index("c")
        s = lax.axis_index("s")
        wid = c * ns + s
        zero16 = jnp.zeros((16,), jnp.float32)

        def zrow(i, carry):
            for j in range(feat // 16):
                buf[i, pl.ds(j * 16, 16)] = zero16
            return carry

        lax.fori_loop(0, 128, zrow, 0)
        for j in range(rows_per_tile // 128):
            pltpu.sync_copy(buf, acc_sh.at[pl.ds(s * rows_per_tile + j * 128, 128)])
        plsc.subcore_barrier()
        pltpu.sync_copy(src_hbm.at[pl.ds(wid * rpw, rpw)], src_v)
        pltpu.sync_copy(dst_hbm.at[pl.ds(wid * rpw, rpw)], dst_v)
        for j in range(rpw):
            pltpu.sync_copy(z_hbm.at[src_v.at[j]], buf)
            pltpu.sync_copy(buf, acc_sh.at[dst_v.at[j]], add=True)
        plsc.subcore_barrier()
        for j in range(rows_per_tile // 128):
            r0 = s * rows_per_tile + j * 128
            pltpu.sync_copy(acc_sh.at[pl.ds(r0, 128)],
                            out_hbm.at[c, pl.ds(r0, 128)])

    return k(z, srcR, dstR)


def _tc_first(x, W, dinv):
    n, d = x.shape
    h = W.shape[1]

    def body(x_ref, w_ref, dinv_ref, z_ref):
        xw = jnp.dot(x_ref[...], w_ref[...], preferred_element_type=jnp.float32)
        z_ref[...] = dinv_ref[...] * xw

    return pl.pallas_call(
        body,
        grid=(n // BLK,),
        in_specs=[
            pl.BlockSpec((BLK, d), lambda i: (i, 0)),
            pl.BlockSpec((d, h), lambda i: (0, 0)),
            pl.BlockSpec((BLK, 1), lambda i: (i, 0)),
        ],
        out_specs=pl.BlockSpec((BLK, h), lambda i: (i, 0)),
        out_shape=jax.ShapeDtypeStruct((n, h), jnp.float32),
    )(x, W, dinv)


def _tc_mid(p, z, b, W, dinv):
    _, n, h = p.shape
    h2 = W.shape[1]

    def body(p_ref, z_ref, b_ref, w_ref, dinv_ref, zo_ref):
        hh = dinv_ref[...] * (p_ref[0] + p_ref[1] + z_ref[...]) + b_ref[...]
        hh = jnp.maximum(hh, 0.0)
        zo_ref[...] = dinv_ref[...] * jnp.dot(
            hh, w_ref[...], preferred_element_type=jnp.float32)

    return pl.pallas_call(
        body,
        grid=(n // BLK,),
        in_specs=[
            pl.BlockSpec((2, BLK, h), lambda i: (0, i, 0)),
            pl.BlockSpec((BLK, h), lambda i: (i, 0)),
            pl.BlockSpec((1, h), lambda i: (0, 0)),
            pl.BlockSpec((h, h2), lambda i: (0, 0)),
            pl.BlockSpec((BLK, 1), lambda i: (i, 0)),
        ],
        out_specs=pl.BlockSpec((BLK, h2), lambda i: (i, 0)),
        out_shape=jax.ShapeDtypeStruct((n, h2), jnp.float32),
    )(p, z, b, W, dinv)


def _tc_final(p, z, b, dinv, batch2, Wl_pad, bl_pad):
    _, n, h = p.shape
    nblk = n // BLK

    def body(p_ref, z_ref, b_ref, dinv_ref, bt_ref, wl_ref, bl_ref,
             out_ref, sums, counts):
        i = pl.program_id(0)

        @pl.when(i == 0)
        def _():
            sums[...] = jnp.zeros_like(sums)
            counts[...] = jnp.zeros_like(counts)

        hh = dinv_ref[...] * (p_ref[0] + p_ref[1] + z_ref[...]) + b_ref[...]
        hh = jnp.maximum(hh, 0.0)
        onehot = (bt_ref[...] == lax.broadcasted_iota(
            jnp.int32, (BLK, GSEG), 1)).astype(jnp.float32)
        dn = (((0,), (0,)), ((), ()))
        sums[...] += lax.dot_general(onehot, hh, dn,
                                     preferred_element_type=jnp.float32)
        counts[...] += lax.dot_general(onehot, jnp.ones((BLK, h), jnp.float32),
                                       dn, preferred_element_type=jnp.float32)

        @pl.when(i == nblk - 1)
        def _():
            pooled = sums[...] / jnp.maximum(counts[...], 1.0)
            out_ref[...] = jnp.dot(pooled, wl_ref[...],
                                   preferred_element_type=jnp.float32) + bl_ref[...]

    return pl.pallas_call(
        body,
        grid=(nblk,),
        in_specs=[
            pl.BlockSpec((2, BLK, h), lambda i: (0, i, 0)),
            pl.BlockSpec((BLK, h), lambda i: (i, 0)),
            pl.BlockSpec((1, h), lambda i: (0, 0)),
            pl.BlockSpec((BLK, 1), lambda i: (i, 0)),
            pl.BlockSpec((BLK, 1), lambda i: (i, 0)),
            pl.BlockSpec((h, 128), lambda i: (0, 0)),
            pl.BlockSpec((1, 128), lambda i: (0, 0)),
        ],
        out_specs=pl.BlockSpec((GSEG, 128), lambda i: (0, 0)),
        out_shape=jax.ShapeDtypeStruct((GSEG, 128), jnp.float32),
        scratch_shapes=[
            pltpu.VMEM((GSEG, h), jnp.float32),
            pltpu.VMEM((GSEG, h), jnp.float32),
        ],
    )(p, z, b, dinv, batch2, Wl_pad, bl_pad)


def kernel(x, edge_index, batch, W1, b1, W2, b2, W3, b3, Wl, bl):
    n, d = x.shape
    e = edge_index.shape[1]
    h = W1.shape[1]

    info = plsc.get_sparse_core_info()
    nc, ns = info.num_cores, info.num_subcores
    nw = nc * ns

    rows = -(-e // 128)
    rpw = -(-rows // nw)
    rows_tot = rpw * nw
    pad_e = rows_tot * 128 - e

    x_pad = jnp.pad(x, ((0, NPAD - n), (0, 0)))
    fill = jnp.full((pad_e,), NPAD - 1, jnp.int32)
    srcR = jnp.concatenate([edge_index[0], fill]).reshape(rows_tot, 128)
    dstR = jnp.concatenate([edge_index[1], fill]).reshape(rows_tot, 128)
    batch2 = jnp.pad(batch, (0, NPAD - n), constant_values=GSEG).reshape(NPAD, 1)
    b1r = b1.reshape(1, h)
    b2r = b2.reshape(1, h)
    b3r = b3.reshape(1, h)
    Wl_pad = jnp.pad(Wl, ((0, 0), (0, 128 - Wl.shape[1])))
    bl_pad = jnp.pad(bl, (0, 128 - bl.shape[0])).reshape(1, 128)

    cnt = _sc_count(dstR, nc, ns, rpw)
    dinv = lax.rsqrt(cnt.sum(axis=0) + 1.0).reshape(NPAD, 1)

    z1 = _tc_first(x_pad, W1, dinv)
    p1 = _sc_scatter(z1, srcR, dstR, nc, ns, rpw, h)
    z2 = _tc_mid(p1, z1, b1r, W2, dinv)
    p2 = _sc_scatter(z2, srcR, dstR, nc, ns, rpw, h)
    z3 = _tc_mid(p2, z2, b2r, W3, dinv)
    p3 = _sc_scatter(z3, srcR, dstR, nc, ns, rpw, h)
    out128 = _tc_final(p3, z3, b3r, dinv, batch2, Wl_pad, bl_pad)
    return out128[:, : Wl.shape[1]]


# speedup vs baseline: 6.9414x; 6.9414x over previous
import functools

import jax
import jax.numpy as jnp
from jax import lax
from jax.experimental import pallas as pl
from jax.experimental.pallas import tpu as pltpu
from jax.experimental.pallas import tpu_sc as plsc

NPAD = 10240
GSEG = 128
BLK = 512


def _sc_count(dstR, nc, ns, rpw):
    mesh = plsc.VectorSubcoreMesh(core_axis_name="c", subcore_axis_name="s")
    rows_per_tile = NPAD // ns

    @functools.partial(
        pl.kernel,
        out_type=jax.ShapeDtypeStruct((nc, NPAD), jnp.float32),
        mesh=mesh,
        scratch_types=[
            pltpu.VMEM((rpw, 128), jnp.int32),
            pltpu.VMEM((128,), jnp.float32),
            pltpu.VMEM_SHARED((NPAD,), jnp.float32),
        ],
    )
    def k(dst_hbm, out_hbm, dst_v, val_v, acc_sh):
        c = lax.axis_index("c")
        s = lax.axis_index("s")
        wid = c * ns + s
        zero16 = jnp.zeros((16,), jnp.float32)
        for j in range(128 // 16):
            val_v[pl.ds(j * 16, 16)] = zero16
        for j in range(rows_per_tile // 128):
            pltpu.sync_copy(val_v, acc_sh.at[pl.ds(s * rows_per_tile + j * 128, 128)])
        plsc.subcore_barrier()
        one16 = jnp.ones((16,), jnp.float32)
        for j in range(128 // 16):
            val_v[pl.ds(j * 16, 16)] = one16
        pltpu.sync_copy(dst_hbm.at[pl.ds(wid * rpw, rpw)], dst_v)
        for j in range(rpw):
            pltpu.sync_copy(val_v, acc_sh.at[dst_v.at[j]], add=True)
        plsc.subcore_barrier()
        pltpu.sync_copy(acc_sh.at[pl.ds(s * rows_per_tile, rows_per_tile)],
                        out_hbm.at[c, pl.ds(s * rows_per_tile, rows_per_tile)])

    return k(dstR)


def _sc_scatter(z, srcR, dstR, nc, ns, rpw, feat):
    mesh = plsc.VectorSubcoreMesh(core_axis_name="c", subcore_axis_name="s")
    rows_per_tile = NPAD // ns

    @functools.partial(
        pl.kernel,
        out_type=jax.ShapeDtypeStruct((nc, NPAD, feat), jnp.float32),
        mesh=mesh,
        scratch_types=[
            pltpu.VMEM((rpw, 128), jnp.int32),
            pltpu.VMEM((rpw, 128), jnp.int32),
            pltpu.VMEM((128, feat), jnp.float32),
            pltpu.VMEM_SHARED((NPAD, feat), jnp.float32),
        ],
    )
    def k(z_hbm, src_hbm, dst_hbm, out_hbm, src_v, dst_v, buf, acc_sh):
        c = lax.axis_index("c")
        s = lax.axis_index("s")
        wid = c * ns + s
        zero16 = jnp.zeros((16,), jnp.float32)

        def zrow(i, carry):
            for j in range(feat // 16):
                buf[i, pl.ds(j * 16, 16)] = zero16
            return carry

        lax.fori_loop(0, 128, zrow, 0)
        for j in range(rows_per_tile // 128):
            pltpu.sync_copy(buf, acc_sh.at[pl.ds(s * rows_per_tile + j * 128, 128)])
        plsc.subcore_barrier()
        pltpu.sync_copy(src_hbm.at[pl.ds(wid * rpw, rpw)], src_v)
        pltpu.sync_copy(dst_hbm.at[pl.ds(wid * rpw, rpw)], dst_v)
        for j in range(rpw):
            pltpu.sync_copy(z_hbm.at[src_v.at[j]], buf)
            pltpu.sync_copy(buf, acc_sh.at[dst_v.at[j]], add=True)
        plsc.subcore_barrier()
        for j in range(rows_per_tile // 128):
            r0 = s * rows_per_tile + j * 128
            pltpu.sync_copy(acc_sh.at[pl.ds(r0, 128)],
                            out_hbm.at[c, pl.ds(r0, 128)])

    return k(z, srcR, dstR)


def _tc_first(x, W, dinv):
    n, d = x.shape
    h = W.shape[1]

    def body(x_ref, w_ref, dinv_ref, z_ref):
        xw = jnp.dot(x_ref[...], w_ref[...], preferred_element_type=jnp.float32)
        z_ref[...] = dinv_ref[...] * xw

    return pl.pallas_call(
        body,
        grid=(n // BLK,),
        in_specs=[
            pl.BlockSpec((BLK, d), lambda i: (i, 0)),
            pl.BlockSpec((d, h), lambda i: (0, 0)),
            pl.BlockSpec((BLK, 1), lambda i: (i, 0)),
        ],
        out_specs=pl.BlockSpec((BLK, h), lambda i: (i, 0)),
        out_shape=jax.ShapeDtypeStruct((n, h), jnp.float32),
    )(x, W, dinv)


def _tc_mid(p, z, b, W, dinv):
    _, n, h = p.shape
    h2 = W.shape[1]

    def body(p_ref, z_ref, b_ref, w_ref, dinv_ref, zo_ref):
        hh = dinv_ref[...] * (p_ref[0] + p_ref[1] + z_ref[...]) + b_ref[...]
        hh = jnp.maximum(hh, 0.0)
        zo_ref[...] = dinv_ref[...] * jnp.dot(
            hh, w_ref[...], preferred_element_type=jnp.float32)

    return pl.pallas_call(
        body,
        grid=(n // BLK,),
        in_specs=[
            pl.BlockSpec((2, BLK, h), lambda i: (0, i, 0)),
            pl.BlockSpec((BLK, h), lambda i: (i, 0)),
            pl.BlockSpec((1, h), lambda i: (0, 0)),
            pl.BlockSpec((h, h2), lambda i: (0, 0)),
            pl.BlockSpec((BLK, 1), lambda i: (i, 0)),
        ],
        out_specs=pl.BlockSpec((BLK, h2), lambda i: (i, 0)),
        out_shape=jax.ShapeDtypeStruct((n, h2), jnp.float32),
    )(p, z, b, W, dinv)


def _tc_final(p, z, b, dinv, batch2, Wl_pad, bl_pad):
    _, n, h = p.shape
    nblk = n // BLK

    def body(p_ref, z_ref, b_ref, dinv_ref, bt_ref, wl_ref, bl_ref,
             out_ref, sums, counts):
        i = pl.program_id(0)

        @pl.when(i == 0)
        def _():
            sums[...] = jnp.zeros_like(sums)
            counts[...] = jnp.zeros_like(counts)

        hh = dinv_ref[...] * (p_ref[0] + p_ref[1] + z_ref[...]) + b_ref[...]
        hh = jnp.maximum(hh, 0.0)
        onehot = (bt_ref[...] == lax.broadcasted_iota(
            jnp.int32, (BLK, GSEG), 1)).astype(jnp.float32)
        dn = (((0,), (0,)), ((), ()))
        sums[...] += lax.dot_general(onehot, hh, dn,
                                     preferred_element_type=jnp.float32)
        counts[...] += lax.dot_general(onehot, jnp.ones((BLK, h), jnp.float32),
                                       dn, preferred_element_type=jnp.float32)

        @pl.when(i == nblk - 1)
        def _():
            pooled = sums[...] / jnp.maximum(counts[...], 1.0)
            out_ref[...] = jnp.dot(pooled, wl_ref[...],
                                   preferred_element_type=jnp.float32) + bl_ref[...]

    return pl.pallas_call(
        body,
        grid=(nblk,),
        in_specs=[
            pl.BlockSpec((2, BLK, h), lambda i: (0, i, 0)),
            pl.BlockSpec((BLK, h), lambda i: (i, 0)),
            pl.BlockSpec((1, h), lambda i: (0, 0)),
            pl.BlockSpec((BLK, 1), lambda i: (i, 0)),
            pl.BlockSpec((BLK, 1), lambda i: (i, 0)),
            pl.BlockSpec((h, 128), lambda i: (0, 0)),
            pl.BlockSpec((1, 128), lambda i: (0, 0)),
        ],
        out_specs=pl.BlockSpec((GSEG, 128), lambda i: (0, 0)),
        out_shape=jax.ShapeDtypeStruct((GSEG, 128), jnp.float32),
        scratch_shapes=[
            pltpu.VMEM((GSEG, h), jnp.float32),
            pltpu.VMEM((GSEG, h), jnp.float32),
        ],
    )(p, z, b, dinv, batch2, Wl_pad, bl_pad)


def kernel(x, edge_index, batch, W1, b1, W2, b2, W3, b3, Wl, bl):
    n, d = x.shape
    e = edge_index.shape[1]
    h = W1.shape[1]

    info = plsc.get_sparse_core_info()
    nc, ns = info.num_cores, info.num_subcores
    nw = nc * ns

    rows = -(-e // 128)
    rpw = -(-rows // nw)
    rpw = -(-rpw // 8) * 8
    rows_tot = rpw * nw
    pad_e = rows_tot * 128 - e

    x_pad = jnp.pad(x, ((0, NPAD - n), (0, 0)))
    fill = jnp.full((pad_e,), NPAD - 1, jnp.int32)
    srcR = jnp.concatenate([edge_index[0], fill]).reshape(rows_tot, 128)
    dstR = jnp.concatenate([edge_index[1], fill]).reshape(rows_tot, 128)
    batch2 = jnp.pad(batch, (0, NPAD - n), constant_values=GSEG).reshape(NPAD, 1)
    b1r = b1.reshape(1, h)
    b2r = b2.reshape(1, h)
    b3r = b3.reshape(1, h)
    Wl_pad = jnp.pad(Wl, ((0, 0), (0, 128 - Wl.shape[1])))
    bl_pad = jnp.pad(bl, (0, 128 - bl.shape[0])).reshape(1, 128)

    cnt = _sc_count(dstR, nc, ns, rpw)
    dinv = lax.rsqrt(cnt.sum(axis=0) + 1.0).reshape(NPAD, 1)

    z1 = _tc_first(x_pad, W1, dinv)
    p1 = _sc_scatter(z1, srcR, dstR, nc, ns, rpw, h)
    z2 = _tc_mid(p1, z1, b1r, W2, dinv)
    p2 = _sc_scatter(z2, srcR, dstR, nc, ns, rpw, h)
    z3 = _tc_mid(p2, z2, b2r, W3, dinv)
    p3 = _sc_scatter(z3, srcR, dstR, nc, ns, rpw, h)
    out128 = _tc_final(p3, z3, b3r, dinv, batch2, Wl_pad, bl_pad)
    return out128[:, : Wl.shape[1]]


# speedup vs baseline: 7.7863x; 1.1217x over previous
import functools

import jax
import jax.numpy as jnp
from jax import lax
from jax.experimental import pallas as pl
from jax.experimental.pallas import tpu as pltpu
from jax.experimental.pallas import tpu_sc as plsc

NPAD = 10240
GSEG = 128
BLK = 512


def _sc_count(dstR, nc, ns, rpw):
    mesh = plsc.VectorSubcoreMesh(core_axis_name="c", subcore_axis_name="s")
    rows_per_tile = NPAD // ns

    @functools.partial(
        pl.kernel,
        out_type=jax.ShapeDtypeStruct((nc, NPAD), jnp.float32),
        mesh=mesh,
        scratch_types=[
            pltpu.VMEM((rpw, 128), jnp.int32),
            pltpu.VMEM((128,), jnp.float32),
            pltpu.VMEM_SHARED((NPAD,), jnp.float32),
        ],
    )
    def k(dst_hbm, out_hbm, dst_v, val_v, acc_sh):
        c = lax.axis_index("c")
        s = lax.axis_index("s")
        wid = c * ns + s
        zero16 = jnp.zeros((16,), jnp.float32)
        for j in range(128 // 16):
            val_v[pl.ds(j * 16, 16)] = zero16
        for j in range(rows_per_tile // 128):
            pltpu.sync_copy(val_v, acc_sh.at[pl.ds(s * rows_per_tile + j * 128, 128)])
        plsc.subcore_barrier()
        one16 = jnp.ones((16,), jnp.float32)
        for j in range(128 // 16):
            val_v[pl.ds(j * 16, 16)] = one16
        pltpu.sync_copy(dst_hbm.at[pl.ds(wid * rpw, rpw)], dst_v)
        for j in range(rpw):
            pltpu.sync_copy(val_v, acc_sh.at[dst_v.at[j]], add=True)
        plsc.subcore_barrier()
        pltpu.sync_copy(acc_sh.at[pl.ds(s * rows_per_tile, rows_per_tile)],
                        out_hbm.at[c, pl.ds(s * rows_per_tile, rows_per_tile)])

    return k(dstR)


def _sc_scatter(z, srcR, dstR, nc, ns, rpw, feat):
    mesh = plsc.VectorSubcoreMesh(core_axis_name="c", subcore_axis_name="s")
    rows_per_tile = NPAD // ns

    nbuf = 2
    pipe = 1
    ich = 16
    nich = rpw // ich
    assert nich * ich == rpw

    @functools.partial(
        pl.kernel,
        out_type=jax.ShapeDtypeStruct((nc, NPAD, feat), jnp.float32),
        mesh=mesh,
        scratch_types=(
            [pltpu.VMEM((ich, 128), jnp.int32) for _ in range(4)]
            + [pltpu.VMEM((128, feat), jnp.float32) for _ in range(nbuf)]
            + [pltpu.VMEM_SHARED((NPAD, feat), jnp.float32)]
            + [pltpu.SemaphoreType.DMA for _ in range(2 * nbuf + 4)]
        ),
    )
    def k(z_hbm, src_hbm, dst_hbm, out_hbm, *rest):
        sidx = rest[0:2]
        didx = rest[2:4]
        bufs = rest[4:4 + nbuf]
        acc_sh = rest[4 + nbuf]
        gsem = rest[5 + nbuf:5 + 2 * nbuf]
        ssem = rest[5 + 2 * nbuf:5 + 3 * nbuf]
        isem = rest[5 + 3 * nbuf:]
        c = lax.axis_index("c")
        s = lax.axis_index("s")
        wid = c * ns + s
        zero16 = jnp.zeros((16,), jnp.float32)
        buf0 = bufs[0]

        def zrow(i, carry):
            for j in range(feat // 16):
                buf0[i, pl.ds(j * 16, 16)] = zero16
            return carry

        lax.fori_loop(0, 128, zrow, 0)
        for j in range(rows_per_tile // 128):
            pltpu.sync_copy(buf0, acc_sh.at[pl.ds(s * rows_per_tile + j * 128, 128)])
        plsc.subcore_barrier()

        def issue_idx(ci, sl):
            r0 = wid * rpw + ci * ich
            return (
                pltpu.async_copy(src_hbm.at[pl.ds(r0, ich)], sidx[sl],
                                 isem[2 * sl]),
                pltpu.async_copy(dst_hbm.at[pl.ds(r0, ich)], didx[sl],
                                 isem[2 * sl + 1]),
            )

        icp = [None, None]
        icp[0] = issue_idx(0, 0)

        gcp = [None] * nbuf
        scp = [None] * nbuf

        for ci in range(nich):
            sl = ci % 2
            for dsc in icp[sl]:
                dsc.wait()
            for r in range(ich):
                j = ci * ich + r
                b = j % nbuf
                if j >= nbuf:
                    scp[b].wait()
                gcp[b] = pltpu.async_copy(
                    z_hbm.at[sidx[sl].at[r]], bufs[b], gsem[b])
                if j >= pipe:
                    i = j - pipe
                    bi = i % nbuf
                    ci_i, r_i = divmod(i, ich)
                    gcp[bi].wait()
                    scp[bi] = pltpu.async_copy(
                        bufs[bi], acc_sh.at[didx[ci_i % 2].at[r_i]],
                        ssem[bi], add=True)
                if r == nbuf + pipe + 1 and ci + 1 < nich:
                    icp[1 - sl] = issue_idx(ci + 1, 1 - sl)
        for i in range(rpw - pipe, rpw):
            bi = i % nbuf
            ci_i, r_i = divmod(i, ich)
            gcp[bi].wait()
            scp[bi] = pltpu.async_copy(
                bufs[bi], acc_sh.at[didx[ci_i % 2].at[r_i]], ssem[bi], add=True)
        for i in range(rpw - nbuf, rpw):
            scp[i % nbuf].wait()
        plsc.subcore_barrier()
        for j in range(rows_per_tile // 128):
            r0 = s * rows_per_tile + j * 128
            pltpu.sync_copy(acc_sh.at[pl.ds(r0, 128)],
                            out_hbm.at[c, pl.ds(r0, 128)])

    return k(z, srcR, dstR)


def _tc_first(x, W, dinv):
    n, d = x.shape
    h = W.shape[1]

    def body(x_ref, w_ref, dinv_ref, z_ref):
        xw = jnp.dot(x_ref[...], w_ref[...], preferred_element_type=jnp.float32)
        z_ref[...] = dinv_ref[...] * xw

    return pl.pallas_call(
        body,
        grid=(n // BLK,),
        in_specs=[
            pl.BlockSpec((BLK, d), lambda i: (i, 0)),
            pl.BlockSpec((d, h), lambda i: (0, 0)),
            pl.BlockSpec((BLK, 1), lambda i: (i, 0)),
        ],
        out_specs=pl.BlockSpec((BLK, h), lambda i: (i, 0)),
        out_shape=jax.ShapeDtypeStruct((n, h), jnp.float32),
    )(x, W, dinv)


def _tc_mid(p, z, b, W, dinv):
    _, n, h = p.shape
    h2 = W.shape[1]

    def body(p_ref, z_ref, b_ref, w_ref, dinv_ref, zo_ref):
        hh = dinv_ref[...] * (p_ref[0] + p_ref[1] + z_ref[...]) + b_ref[...]
        hh = jnp.maximum(hh, 0.0)
        zo_ref[...] = dinv_ref[...] * jnp.dot(
            hh, w_ref[...], preferred_element_type=jnp.float32)

    return pl.pallas_call(
        body,
        grid=(n // BLK,),
        in_specs=[
            pl.BlockSpec((2, BLK, h), lambda i: (0, i, 0)),
            pl.BlockSpec((BLK, h), lambda i: (i, 0)),
            pl.BlockSpec((1, h), lambda i: (0, 0)),
            pl.BlockSpec((h, h2), lambda i: (0, 0)),
            pl.BlockSpec((BLK, 1), lambda i: (i, 0)),
        ],
        out_specs=pl.BlockSpec((BLK, h2), lambda i: (i, 0)),
        out_shape=jax.ShapeDtypeStruct((n, h2), jnp.float32),
    )(p, z, b, W, dinv)


def _tc_final(p, z, b, dinv, batch2, Wl_pad, bl_pad):
    _, n, h = p.shape
    nblk = n // BLK

    def body(p_ref, z_ref, b_ref, dinv_ref, bt_ref, wl_ref, bl_ref,
             out_ref, sums, counts):
        i = pl.program_id(0)

        @pl.when(i == 0)
        def _():
            sums[...] = jnp.zeros_like(sums)
            counts[...] = jnp.zeros_like(counts)

        hh = dinv_ref[...] * (p_ref[0] + p_ref[1] + z_ref[...]) + b_ref[...]
        hh = jnp.maximum(hh, 0.0)
        onehot = (bt_ref[...] == lax.broadcasted_iota(
            jnp.int32, (BLK, GSEG), 1)).astype(jnp.float32)
        dn = (((0,), (0,)), ((), ()))
        sums[...] += lax.dot_general(onehot, hh, dn,
                                     preferred_element_type=jnp.float32)
        counts[...] += lax.dot_general(onehot, jnp.ones((BLK, h), jnp.float32),
                                       dn, preferred_element_type=jnp.float32)

        @pl.when(i == nblk - 1)
        def _():
            pooled = sums[...] / jnp.maximum(counts[...], 1.0)
            out_ref[...] = jnp.dot(pooled, wl_ref[...],
                                   preferred_element_type=jnp.float32) + bl_ref[...]

    return pl.pallas_call(
        body,
        grid=(nblk,),
        in_specs=[
            pl.BlockSpec((2, BLK, h), lambda i: (0, i, 0)),
            pl.BlockSpec((BLK, h), lambda i: (i, 0)),
            pl.BlockSpec((1, h), lambda i: (0, 0)),
            pl.BlockSpec((BLK, 1), lambda i: (i, 0)),
            pl.BlockSpec((BLK, 1), lambda i: (i, 0)),
            pl.BlockSpec((h, 128), lambda i: (0, 0)),
            pl.BlockSpec((1, 128), lambda i: (0, 0)),
        ],
        out_specs=pl.BlockSpec((GSEG, 128), lambda i: (0, 0)),
        out_shape=jax.ShapeDtypeStruct((GSEG, 128), jnp.float32),
        scratch_shapes=[
            pltpu.VMEM((GSEG, h), jnp.float32),
            pltpu.VMEM((GSEG, h), jnp.float32),
        ],
    )(p, z, b, dinv, batch2, Wl_pad, bl_pad)


def kernel(x, edge_index, batch, W1, b1, W2, b2, W3, b3, Wl, bl):
    n, d = x.shape
    e = edge_index.shape[1]
    h = W1.shape[1]

    info = plsc.get_sparse_core_info()
    nc, ns = info.num_cores, info.num_subcores
    nw = nc * ns

    rows = -(-e // 128)
    rpw = -(-rows // nw)
    rpw = -(-rpw // 16) * 16
    rows_tot = rpw * nw
    pad_e = rows_tot * 128 - e

    x_pad = jnp.pad(x, ((0, NPAD - n), (0, 0)))
    fill = jnp.full((pad_e,), NPAD - 1, jnp.int32)
    srcR = jnp.concatenate([edge_index[0], fill]).reshape(rows_tot, 128)
    dstR = jnp.concatenate([edge_index[1], fill]).reshape(rows_tot, 128)
    batch2 = jnp.pad(batch, (0, NPAD - n), constant_values=GSEG).reshape(NPAD, 1)
    b1r = b1.reshape(1, h)
    b2r = b2.reshape(1, h)
    b3r = b3.reshape(1, h)
    Wl_pad = jnp.pad(Wl, ((0, 0), (0, 128 - Wl.shape[1])))
    bl_pad = jnp.pad(bl, (0, 128 - bl.shape[0])).reshape(1, 128)

    cnt = _sc_count(dstR, nc, ns, rpw)
    dinv = lax.rsqrt(cnt.sum(axis=0) + 1.0).reshape(NPAD, 1)

    z1 = _tc_first(x_pad, W1, dinv)
    p1 = _sc_scatter(z1, srcR, dstR, nc, ns, rpw, h)
    z2 = _tc_mid(p1, z1, b1r, W2, dinv)
    p2 = _sc_scatter(z2, srcR, dstR, nc, ns, rpw, h)
    z3 = _tc_mid(p2, z2, b2r, W3, dinv)
    p3 = _sc_scatter(z3, srcR, dstR, nc, ns, rpw, h)
    out128 = _tc_final(p3, z3, b3r, dinv, batch2, Wl_pad, bl_pad)
    return out128[:, : Wl.shape[1]]


# speedup vs baseline: 9.1315x; 1.1728x over previous
import functools

import jax
import jax.numpy as jnp
from jax import lax
from jax.experimental import pallas as pl
from jax.experimental.pallas import tpu as pltpu
from jax.experimental.pallas import tpu_sc as plsc

NPAD = 10240
GSEG = 128
BLK = 512


def _sc_count(dstR, nc, ns, rpw):
    mesh = plsc.VectorSubcoreMesh(core_axis_name="c", subcore_axis_name="s")
    rows_per_tile = NPAD // ns

    @functools.partial(
        pl.kernel,
        out_type=jax.ShapeDtypeStruct((nc, NPAD), jnp.float32),
        mesh=mesh,
        scratch_types=[
            pltpu.VMEM((rpw, 128), jnp.int32),
            pltpu.VMEM((128,), jnp.float32),
            pltpu.VMEM_SHARED((NPAD,), jnp.float32),
        ],
    )
    def k(dst_hbm, out_hbm, dst_v, val_v, acc_sh):
        c = lax.axis_index("c")
        s = lax.axis_index("s")
        wid = c * ns + s
        zero16 = jnp.zeros((16,), jnp.float32)
        for j in range(128 // 16):
            val_v[pl.ds(j * 16, 16)] = zero16
        for j in range(rows_per_tile // 128):
            pltpu.sync_copy(val_v, acc_sh.at[pl.ds(s * rows_per_tile + j * 128, 128)])
        plsc.subcore_barrier()
        one16 = jnp.ones((16,), jnp.float32)
        for j in range(128 // 16):
            val_v[pl.ds(j * 16, 16)] = one16
        pltpu.sync_copy(dst_hbm.at[pl.ds(wid * rpw, rpw)], dst_v)
        for j in range(rpw):
            pltpu.sync_copy(val_v, acc_sh.at[dst_v.at[j]], add=True)
        plsc.subcore_barrier()
        pltpu.sync_copy(acc_sh.at[pl.ds(s * rows_per_tile, rows_per_tile)],
                        out_hbm.at[c, pl.ds(s * rows_per_tile, rows_per_tile)])

    return k(dstR)


def _sc_scatter(z, srcR, dstR, nc, ns, rpw, feat):
    mesh = plsc.VectorSubcoreMesh(core_axis_name="c", subcore_axis_name="s")
    rows_per_tile = NPAD // ns

    nbuf = 2
    pipe = 1
    ich = 16
    rows_tot = rpw * nc * ns
    rows_pair = rows_tot // ns
    r_core = [0, 0]
    r_core[0] = min(rows_pair - ich, max(ich, (rows_pair * 4 // 5) // ich * ich))
    r_core[1] = rows_pair - r_core[0]

    @functools.partial(
        pl.kernel,
        out_type=jax.ShapeDtypeStruct((nc, NPAD, feat), jnp.float32),
        mesh=mesh,
        scratch_types=(
            [pltpu.VMEM((ich, 128), jnp.int32) for _ in range(4)]
            + [pltpu.VMEM((128, feat), jnp.float32) for _ in range(nbuf)]
            + [pltpu.VMEM_SHARED((NPAD, feat), jnp.float32)]
            + [pltpu.SemaphoreType.DMA for _ in range(2 * nbuf + 4)]
        ),
    )
    def k(z_hbm, src_hbm, dst_hbm, out_hbm, *rest):
        sidx = rest[0:2]
        didx = rest[2:4]
        bufs = rest[4:4 + nbuf]
        acc_sh = rest[4 + nbuf]
        gsem = rest[5 + nbuf:5 + 2 * nbuf]
        ssem = rest[5 + 2 * nbuf:5 + 3 * nbuf]
        isem = rest[5 + 3 * nbuf:]
        c = lax.axis_index("c")
        s = lax.axis_index("s")
        zero16 = jnp.zeros((16,), jnp.float32)
        buf0 = bufs[0]

        def zrow(i, carry):
            for j in range(feat // 16):
                buf0[i, pl.ds(j * 16, 16)] = zero16
            return carry

        lax.fori_loop(0, 128, zrow, 0)
        for j in range(rows_per_tile // 128):
            pltpu.sync_copy(buf0, acc_sh.at[pl.ds(s * rows_per_tile + j * 128, 128)])
        plsc.subcore_barrier()

        def edge_pipeline(row_base, nrows):
            nich = nrows // ich

            def issue_idx(ci, sl):
                r0 = row_base + ci * ich
                return (
                    pltpu.async_copy(src_hbm.at[pl.ds(r0, ich)], sidx[sl],
                                     isem[2 * sl]),
                    pltpu.async_copy(dst_hbm.at[pl.ds(r0, ich)], didx[sl],
                                     isem[2 * sl + 1]),
                )

            icp = [None, None]
            icp[0] = issue_idx(0, 0)
            gcp = [None] * nbuf
            scp = [None] * nbuf

            for ci in range(nich):
                sl = ci % 2
                for dsc in icp[sl]:
                    dsc.wait()
                for r in range(ich):
                    j = ci * ich + r
                    b = j % nbuf
                    if j >= nbuf:
                        scp[b].wait()
                    gcp[b] = pltpu.async_copy(
                        z_hbm.at[sidx[sl].at[r]], bufs[b], gsem[b])
                    if j >= pipe:
                        i = j - pipe
                        bi = i % nbuf
                        ci_i, r_i = divmod(i, ich)
                        gcp[bi].wait()
                        scp[bi] = pltpu.async_copy(
                            bufs[bi], acc_sh.at[didx[ci_i % 2].at[r_i]],
                            ssem[bi], add=True)
                    if r == nbuf + pipe + 1 and ci + 1 < nich:
                        icp[1 - sl] = issue_idx(ci + 1, 1 - sl)
            for i in range(nrows - pipe, nrows):
                bi = i % nbuf
                ci_i, r_i = divmod(i, ich)
                gcp[bi].wait()
                scp[bi] = pltpu.async_copy(
                    bufs[bi], acc_sh.at[didx[ci_i % 2].at[r_i]],
                    ssem[bi], add=True)
            for i in range(max(nrows - nbuf, 0), nrows):
                scp[i % nbuf].wait()

        @pl.when(c == 0)
        def _():
            edge_pipeline(s * rows_pair, r_core[0])

        @pl.when(c == 1)
        def _():
            edge_pipeline(s * rows_pair + r_core[0], r_core[1])

        plsc.subcore_barrier()
        for j in range(rows_per_tile // 128):
            r0 = s * rows_per_tile + j * 128
            pltpu.sync_copy(acc_sh.at[pl.ds(r0, 128)],
                            out_hbm.at[c, pl.ds(r0, 128)])

    return k(z, srcR, dstR)


def _tc_first(x, W, dinv):
    n, d = x.shape
    h = W.shape[1]

    def body(x_ref, w_ref, dinv_ref, z_ref):
        xw = jnp.dot(x_ref[...], w_ref[...], preferred_element_type=jnp.float32)
        z_ref[...] = dinv_ref[...] * xw

    return pl.pallas_call(
        body,
        grid=(n // BLK,),
        in_specs=[
            pl.BlockSpec((BLK, d), lambda i: (i, 0)),
            pl.BlockSpec((d, h), lambda i: (0, 0)),
            pl.BlockSpec((BLK, 1), lambda i: (i, 0)),
        ],
        out_specs=pl.BlockSpec((BLK, h), lambda i: (i, 0)),
        out_shape=jax.ShapeDtypeStruct((n, h), jnp.float32),
    )(x, W, dinv)


def _tc_mid(p, z, b, W, dinv):
    _, n, h = p.shape
    h2 = W.shape[1]

    def body(p_ref, z_ref, b_ref, w_ref, dinv_ref, zo_ref):
        hh = dinv_ref[...] * (p_ref[0] + p_ref[1] + z_ref[...]) + b_ref[...]
        hh = jnp.maximum(hh, 0.0)
        zo_ref[...] = dinv_ref[...] * jnp.dot(
            hh, w_ref[...], preferred_element_type=jnp.float32)

    return pl.pallas_call(
        body,
        grid=(n // BLK,),
        in_specs=[
            pl.BlockSpec((2, BLK, h), lambda i: (0, i, 0)),
            pl.BlockSpec((BLK, h), lambda i: (i, 0)),
            pl.BlockSpec((1, h), lambda i: (0, 0)),
            pl.BlockSpec((h, h2), lambda i: (0, 0)),
            pl.BlockSpec((BLK, 1), lambda i: (i, 0)),
        ],
        out_specs=pl.BlockSpec((BLK, h2), lambda i: (i, 0)),
        out_shape=jax.ShapeDtypeStruct((n, h2), jnp.float32),
    )(p, z, b, W, dinv)


def _tc_final(p, z, b, dinv, batch2, Wl_pad, bl_pad):
    _, n, h = p.shape
    nblk = n // BLK

    def body(p_ref, z_ref, b_ref, dinv_ref, bt_ref, wl_ref, bl_ref,
             out_ref, sums, counts):
        i = pl.program_id(0)

        @pl.when(i == 0)
        def _():
            sums[...] = jnp.zeros_like(sums)
            counts[...] = jnp.zeros_like(counts)

        hh = dinv_ref[...] * (p_ref[0] + p_ref[1] + z_ref[...]) + b_ref[...]
        hh = jnp.maximum(hh, 0.0)
        onehot = (bt_ref[...] == lax.broadcasted_iota(
            jnp.int32, (BLK, GSEG), 1)).astype(jnp.float32)
        dn = (((0,), (0,)), ((), ()))
        sums[...] += lax.dot_general(onehot, hh, dn,
                                     preferred_element_type=jnp.float32)
        counts[...] += lax.dot_general(onehot, jnp.ones((BLK, h), jnp.float32),
                                       dn, preferred_element_type=jnp.float32)

        @pl.when(i == nblk - 1)
        def _():
            pooled = sums[...] / jnp.maximum(counts[...], 1.0)
            out_ref[...] = jnp.dot(pooled, wl_ref[...],
                                   preferred_element_type=jnp.float32) + bl_ref[...]

    return pl.pallas_call(
        body,
        grid=(nblk,),
        in_specs=[
            pl.BlockSpec((2, BLK, h), lambda i: (0, i, 0)),
            pl.BlockSpec((BLK, h), lambda i: (i, 0)),
            pl.BlockSpec((1, h), lambda i: (0, 0)),
            pl.BlockSpec((BLK, 1), lambda i: (i, 0)),
            pl.BlockSpec((BLK, 1), lambda i: (i, 0)),
            pl.BlockSpec((h, 128), lambda i: (0, 0)),
            pl.BlockSpec((1, 128), lambda i: (0, 0)),
        ],
        out_specs=pl.BlockSpec((GSEG, 128), lambda i: (0, 0)),
        out_shape=jax.ShapeDtypeStruct((GSEG, 128), jnp.float32),
        scratch_shapes=[
            pltpu.VMEM((GSEG, h), jnp.float32),
            pltpu.VMEM((GSEG, h), jnp.float32),
        ],
    )(p, z, b, dinv, batch2, Wl_pad, bl_pad)


def kernel(x, edge_index, batch, W1, b1, W2, b2, W3, b3, Wl, bl):
    n, d = x.shape
    e = edge_index.shape[1]
    h = W1.shape[1]

    info = plsc.get_sparse_core_info()
    nc, ns = info.num_cores, info.num_subcores
    nw = nc * ns

    rows = -(-e // 128)
    rpw = -(-rows // nw)
    rpw = -(-rpw // 16) * 16
    rows_tot = rpw * nw
    pad_e = rows_tot * 128 - e

    x_pad = jnp.pad(x, ((0, NPAD - n), (0, 0)))
    fill = jnp.full((pad_e,), NPAD - 1, jnp.int32)
    srcR = jnp.concatenate([edge_index[0], fill]).reshape(rows_tot, 128)
    dstR = jnp.concatenate([edge_index[1], fill]).reshape(rows_tot, 128)
    batch2 = jnp.pad(batch, (0, NPAD - n), constant_values=GSEG).reshape(NPAD, 1)
    b1r = b1.reshape(1, h)
    b2r = b2.reshape(1, h)
    b3r = b3.reshape(1, h)
    Wl_pad = jnp.pad(Wl, ((0, 0), (0, 128 - Wl.shape[1])))
    bl_pad = jnp.pad(bl, (0, 128 - bl.shape[0])).reshape(1, 128)

    cnt = _sc_count(dstR, nc, ns, rpw)
    dinv = lax.rsqrt(cnt.sum(axis=0) + 1.0).reshape(NPAD, 1)

    z1 = _tc_first(x_pad, W1, dinv)
    p1 = _sc_scatter(z1, srcR, dstR, nc, ns, rpw, h)
    z2 = _tc_mid(p1, z1, b1r, W2, dinv)
    p2 = _sc_scatter(z2, srcR, dstR, nc, ns, rpw, h)
    z3 = _tc_mid(p2, z2, b2r, W3, dinv)
    p3 = _sc_scatter(z3, srcR, dstR, nc, ns, rpw, h)
    out128 = _tc_final(p3, z3, b3r, dinv, batch2, Wl_pad, bl_pad)
    return out128[:, : Wl.shape[1]]
